# baseline (speedup 1.0000x reference)
import jax
import jax.numpy as jnp
from jax import lax
from jax.experimental import pallas as pl
from jax.experimental.pallas import tpu as pltpu
from jax.experimental.pallas import tpu_sc as plsc

_B = 4096
_DIM = 64
_MU = 0.5
_NC = 2
_NS = 16
_NW = _NC * _NS
_BPW = _B // _NW
_L = 16
_GROUPS = _BPW // _L


def _mf_body(u_tab, f_tab, ub_tab, fb_tab, users, feeds, out,
             uidx, fidx, urows, frows, ubv, fbv, outv, sem):
    wid = lax.axis_index("s") * _NC + lax.axis_index("c")
    base = wid * _BPW

    pltpu.sync_copy(users.at[pl.ds(base, _BPW)], uidx)
    pltpu.sync_copy(feeds.at[pl.ds(base, _BPW)], fidx)

    copies = [
        pltpu.async_copy(ub_tab.at[uidx], ubv, sem),
        pltpu.async_copy(fb_tab.at[fidx], fbv, sem),
    ]
    for g in range(_GROUPS):
        uv = uidx[pl.ds(g * _L, _L)]
        fv = fidx[pl.ds(g * _L, _L)]
        for j in range(_L):
            r = g * _L + j
            copies.append(pltpu.async_copy(u_tab.at[uv[j]], urows.at[r], sem))
            copies.append(pltpu.async_copy(f_tab.at[fv[j]], frows.at[r], sem))
    for c in copies:
        c.wait()

    lanes = lax.iota(jnp.int32, _L)

    def group(g, carry):
        r0 = g * _L
        rids = r0 + lanes
        acc = ubv[pl.ds(r0, _L)] + fbv[pl.ds(r0, _L)] + _MU
        for d in range(_DIM):
            dsplat = jnp.full((_L,), d, jnp.int32)
            acc = acc + (plsc.load_gather(urows, [rids, dsplat])
                         * plsc.load_gather(frows, [rids, dsplat]))
        outv[pl.ds(r0, _L)] = acc
        return carry

    lax.fori_loop(0, _GROUPS, group, 0)

    pltpu.sync_copy(outv, out.at[pl.ds(base, _BPW)])


def kernel(user_latent, feed_latent, user_bias, feed_bias, users, feeds):
    f = pl.kernel(
        _mf_body,
        out_type=jax.ShapeDtypeStruct((_B,), jnp.float32),
        mesh=plsc.VectorSubcoreMesh(core_axis_name="c", subcore_axis_name="s"),
        compiler_params=pltpu.CompilerParams(needs_layout_passes=False),
        scratch_types=[
            pltpu.VMEM((_BPW,), jnp.int32),
            pltpu.VMEM((_BPW,), jnp.int32),
            pltpu.VMEM((_BPW, _DIM), jnp.float32),
            pltpu.VMEM((_BPW, _DIM), jnp.float32),
            pltpu.VMEM((_BPW,), jnp.float32),
            pltpu.VMEM((_BPW,), jnp.float32),
            pltpu.VMEM((_BPW,), jnp.float32),
            pltpu.SemaphoreType.DMA,
        ],
    )
    return f(user_latent, feed_latent,
             user_bias.reshape(-1), feed_bias.reshape(-1),
             users.astype(jnp.int32), feeds.astype(jnp.int32))


# speedup vs baseline: 3.0136x; 3.0136x over previous
import jax
import jax.numpy as jnp
from jax import lax
from jax.experimental import pallas as pl
from jax.experimental.pallas import tpu as pltpu
from jax.experimental.pallas import tpu_sc as plsc

_B = 4096
_DIM = 64
_MU = 0.5
_NC = 2
_NS = 16
_NW = _NC * _NS
_BPW = _B // _NW
_L = 16
_GROUPS = _BPW // _L
_TW = 128
_N = 1000000
_TAIL = (_N // _TW) * _TW


def _mf_body(u_tab, f_tab, ub_tab, fb_tab, users, feeds, out,
             uidx, fidx, ubuf0, ubuf1, fbuf0, fbuf1, utail, ftail,
             urows, frows, ubv, fbv, outv, sem0, sem1, semb):
    wid = lax.axis_index("s") * _NC + lax.axis_index("c")
    base = wid * _BPW

    pltpu.sync_copy(users.at[pl.ds(base, _BPW)], uidx)
    pltpu.sync_copy(feeds.at[pl.ds(base, _BPW)], fidx)

    bias_copies = [
        pltpu.async_copy(ub_tab.at[uidx], ubv, semb),
        pltpu.async_copy(fb_tab.at[fidx], fbv, semb),
    ]

    pltpu.sync_copy(u_tab.at[:, pl.ds(_TAIL, _N - _TAIL)], utail)
    pltpu.sync_copy(f_tab.at[:, pl.ds(_TAIL, _N - _TAIL)], ftail)

    lanes = lax.iota(jnp.int32, _L)
    ubufs = (ubuf0, ubuf1)
    fbufs = (fbuf0, fbuf1)
    sems = (sem0, sem1)

    def fire(tab, bufs, r_scalar, slot):
        rc = lax.min(r_scalar, _TAIL - 1)
        tile = pl.multiple_of((rc >> 7) << 7, _TW)
        return pltpu.async_copy(tab.at[:, pl.ds(tile, _TW)], bufs[slot],
                                sems[slot])

    uvals = []
    fvals = []
    for g in range(_GROUPS):
        uv = uidx[pl.ds(g * _L, _L)]
        fv = fidx[pl.ds(g * _L, _L)]
        for j in range(_L):
            uvals.append(uv[j])
            fvals.append(fv[j])

    pend_u = [None, None]
    pend_f = [None, None]
    pend_u[0] = fire(u_tab, ubufs, uvals[0], 0)
    pend_f[0] = fire(f_tab, fbufs, fvals[0], 0)

    for k in range(_BPW):
        slot = k & 1
        if k + 1 < _BPW:
            nslot = (k + 1) & 1
            pend_u[nslot] = fire(u_tab, ubufs, uvals[k + 1], nslot)
            pend_f[nslot] = fire(f_tab, fbufs, fvals[k + 1], nslot)
        pend_u[slot].wait()
        pend_f[slot].wait()
        ucs = jnp.full((_L,), lax.min(uvals[k], _TAIL - 1) & (_TW - 1),
                       jnp.int32)
        fcs = jnp.full((_L,), lax.min(fvals[k], _TAIL - 1) & (_TW - 1),
                       jnp.int32)
        for c in range(_DIM // _L):
            dl = c * _L + lanes
            urows[k, pl.ds(c * _L, _L)] = plsc.load_gather(ubufs[slot],
                                                           [dl, ucs])
            frows[k, pl.ds(c * _L, _L)] = plsc.load_gather(fbufs[slot],
                                                           [dl, fcs])

    def patch(k, carry):
        ksplat = jnp.full((_L,), k, jnp.int32)
        for rows, idx_ref, tail in ((urows, uidx, utail),
                                    (frows, fidx, ftail)):
            rv = plsc.load_gather(idx_ref, [ksplat])
            mask = rv >= _TAIL
            tcol = jnp.where(mask, rv - _TAIL, 0)
            for c in range(_DIM // _L):
                dl = c * _L + lanes
                cur = rows[k, pl.ds(c * _L, _L)]
                tv = plsc.load_gather(tail, [dl, tcol])
                rows[k, pl.ds(c * _L, _L)] = jnp.where(mask, tv, cur)
        return carry

    lax.fori_loop(0, _BPW, patch, 0)

    for c in bias_copies:
        c.wait()

    def group(g, carry):
        r0 = g * _L
        rids = r0 + lanes
        acc = ubv[pl.ds(r0, _L)] + fbv[pl.ds(r0, _L)] + _MU
        for d in range(_DIM):
            dsplat = jnp.full((_L,), d, jnp.int32)
            acc = acc + (plsc.load_gather(urows, [rids, dsplat])
                         * plsc.load_gather(frows, [rids, dsplat]))
        outv[pl.ds(r0, _L)] = acc
        return carry

    lax.fori_loop(0, _GROUPS, group, 0)

    pltpu.sync_copy(outv, out.at[pl.ds(base, _BPW)])


def kernel(user_latent, feed_latent, user_bias, feed_bias, users, feeds):
    f = pl.kernel(
        _mf_body,
        out_type=jax.ShapeDtypeStruct((_B,), jnp.float32),
        mesh=plsc.VectorSubcoreMesh(core_axis_name="c", subcore_axis_name="s"),
        compiler_params=pltpu.CompilerParams(needs_layout_passes=False),
        scratch_types=[
            pltpu.VMEM((_BPW,), jnp.int32),
            pltpu.VMEM((_BPW,), jnp.int32),
            pltpu.VMEM((_DIM, _TW), jnp.float32),
            pltpu.VMEM((_DIM, _TW), jnp.float32),
            pltpu.VMEM((_DIM, _TW), jnp.float32),
            pltpu.VMEM((_DIM, _TW), jnp.float32),
            pltpu.VMEM((_DIM, _N - _TAIL), jnp.float32),
            pltpu.VMEM((_DIM, _N - _TAIL), jnp.float32),
            pltpu.VMEM((_BPW, _DIM), jnp.float32),
            pltpu.VMEM((_BPW, _DIM), jnp.float32),
            pltpu.VMEM((_BPW,), jnp.float32),
            pltpu.VMEM((_BPW,), jnp.float32),
            pltpu.VMEM((_BPW,), jnp.float32),
            pltpu.SemaphoreType.DMA,
            pltpu.SemaphoreType.DMA,
            pltpu.SemaphoreType.DMA,
        ],
    )
    return f(user_latent.T, feed_latent.T,
             user_bias.reshape(-1), feed_bias.reshape(-1),
             users.astype(jnp.int32), feeds.astype(jnp.int32))


# speedup vs baseline: 3.3164x; 1.1005x over previous
import jax
import jax.numpy as jnp
from jax import lax
from jax.experimental import pallas as pl
from jax.experimental.pallas import tpu as pltpu
from jax.experimental.pallas import tpu_sc as plsc

_B = 4096
_DIM = 64
_MU = 0.5
_NC = 2
_NS = 16
_NW = _NC * _NS
_BPW = _B // _NW
_L = 16
_GROUPS = _BPW // _L
_TW = 128
_N = 1000000
_TAIL = (_N // _TW) * _TW


def _mf_body(u_tab, f_tab, ub_tab, fb_tab, users, feeds, out,
             uidx, fidx, ubuf0, ubuf1, ubuf2, ubuf3,
             fbuf0, fbuf1, fbuf2, fbuf3, utail, ftail,
             urows, frows, ubv, fbv, outv,
             sem0, sem1, sem2, sem3, semb):
    wid = lax.axis_index("s") * _NC + lax.axis_index("c")
    base = wid * _BPW

    pltpu.sync_copy(users.at[pl.ds(base, _BPW)], uidx)
    pltpu.sync_copy(feeds.at[pl.ds(base, _BPW)], fidx)

    bias_copies = [
        pltpu.async_copy(ub_tab.at[uidx], ubv, semb),
        pltpu.async_copy(fb_tab.at[fidx], fbv, semb),
    ]

    pltpu.sync_copy(u_tab.at[:, pl.ds(_TAIL, _N - _TAIL)], utail)
    pltpu.sync_copy(f_tab.at[:, pl.ds(_TAIL, _N - _TAIL)], ftail)

    lanes = lax.iota(jnp.int32, _L)
    ubufs = (ubuf0, ubuf1, ubuf2, ubuf3)
    fbufs = (fbuf0, fbuf1, fbuf2, fbuf3)
    sems = (sem0, sem1, sem2, sem3)

    def fire(tab, bufs, r_scalar, slot):
        rc = lax.min(r_scalar, _TAIL - 1)
        tile = pl.multiple_of((rc >> 7) << 7, _TW)
        return pltpu.async_copy(tab.at[:, pl.ds(tile, _TW)], bufs[slot],
                                sems[slot])

    uvals = []
    fvals = []
    for g in range(_GROUPS):
        uv = uidx[pl.ds(g * _L, _L)]
        fv = fidx[pl.ds(g * _L, _L)]
        for j in range(_L):
            uvals.append(uv[j])
            fvals.append(fv[j])

    _DEPTH = 4
    pend_u = [None] * _DEPTH
    pend_f = [None] * _DEPTH
    for k0 in range(_DEPTH - 1):
        pend_u[k0] = fire(u_tab, ubufs, uvals[k0], k0)
        pend_f[k0] = fire(f_tab, fbufs, fvals[k0], k0)

    for k in range(_BPW):
        slot = k % _DEPTH
        if k + _DEPTH - 1 < _BPW:
            nslot = (k + _DEPTH - 1) % _DEPTH
            pend_u[nslot] = fire(u_tab, ubufs, uvals[k + _DEPTH - 1], nslot)
            pend_f[nslot] = fire(f_tab, fbufs, fvals[k + _DEPTH - 1], nslot)
        pend_u[slot].wait()
        pend_f[slot].wait()
        ucs = jnp.full((_L,), lax.min(uvals[k], _TAIL - 1) & (_TW - 1),
                       jnp.int32)
        fcs = jnp.full((_L,), lax.min(fvals[k], _TAIL - 1) & (_TW - 1),
                       jnp.int32)
        for c in range(_DIM // _L):
            dl = c * _L + lanes
            urows[k, pl.ds(c * _L, _L)] = plsc.load_gather(ubufs[slot],
                                                           [dl, ucs])
            frows[k, pl.ds(c * _L, _L)] = plsc.load_gather(fbufs[slot],
                                                           [dl, fcs])

    def patch(k, carry):
        ksplat = jnp.full((_L,), k, jnp.int32)
        for rows, idx_ref, tail in ((urows, uidx, utail),
                                    (frows, fidx, ftail)):
            rv = plsc.load_gather(idx_ref, [ksplat])
            mask = rv >= _TAIL
            tcol = jnp.where(mask, rv - _TAIL, 0)
            for c in range(_DIM // _L):
                dl = c * _L + lanes
                cur = rows[k, pl.ds(c * _L, _L)]
                tv = plsc.load_gather(tail, [dl, tcol])
                rows[k, pl.ds(c * _L, _L)] = jnp.where(mask, tv, cur)
        return carry

    lax.fori_loop(0, _BPW, patch, 0)

    for c in bias_copies:
        c.wait()

    def group(g, carry):
        r0 = g * _L
        rids = r0 + lanes
        acc = ubv[pl.ds(r0, _L)] + fbv[pl.ds(r0, _L)] + _MU
        for d in range(_DIM):
            dsplat = jnp.full((_L,), d, jnp.int32)
            acc = acc + (plsc.load_gather(urows, [rids, dsplat])
                         * plsc.load_gather(frows, [rids, dsplat]))
        outv[pl.ds(r0, _L)] = acc
        return carry

    lax.fori_loop(0, _GROUPS, group, 0)

    pltpu.sync_copy(outv, out.at[pl.ds(base, _BPW)])


def kernel(user_latent, feed_latent, user_bias, feed_bias, users, feeds):
    f = pl.kernel(
        _mf_body,
        out_type=jax.ShapeDtypeStruct((_B,), jnp.float32),
        mesh=plsc.VectorSubcoreMesh(core_axis_name="c", subcore_axis_name="s"),
        compiler_params=pltpu.CompilerParams(needs_layout_passes=False),
        scratch_types=[
            pltpu.VMEM((_BPW,), jnp.int32),
            pltpu.VMEM((_BPW,), jnp.int32),
            pltpu.VMEM((_DIM, _TW), jnp.float32),
            pltpu.VMEM((_DIM, _TW), jnp.float32),
            pltpu.VMEM((_DIM, _TW), jnp.float32),
            pltpu.VMEM((_DIM, _TW), jnp.float32),
            pltpu.VMEM((_DIM, _TW), jnp.float32),
            pltpu.VMEM((_DIM, _TW), jnp.float32),
            pltpu.VMEM((_DIM, _TW), jnp.float32),
            pltpu.VMEM((_DIM, _TW), jnp.float32),
            pltpu.VMEM((_DIM, _N - _TAIL), jnp.float32),
            pltpu.VMEM((_DIM, _N - _TAIL), jnp.float32),
            pltpu.VMEM((_BPW, _DIM), jnp.float32),
            pltpu.VMEM((_BPW, _DIM), jnp.float32),
            pltpu.VMEM((_BPW,), jnp.float32),
            pltpu.VMEM((_BPW,), jnp.float32),
            pltpu.VMEM((_BPW,), jnp.float32),
            pltpu.SemaphoreType.DMA,
            pltpu.SemaphoreType.DMA,
            pltpu.SemaphoreType.DMA,
            pltpu.SemaphoreType.DMA,
            pltpu.SemaphoreType.DMA,
        ],
    )
    return f(user_latent.T, feed_latent.T,
             user_bias.reshape(-1), feed_bias.reshape(-1),
             users.astype(jnp.int32), feeds.astype(jnp.int32))
